# baseline (speedup 1.0000x reference)
import functools

import jax
import jax.numpy as jnp
from jax.experimental import pallas as pl
from jax.experimental.pallas import tpu as pltpu

_VMEM_LIMIT = 64 * 1024 * 1024
_BN = 16


def _zero_halo(ref, H, W):
    B, Hp, Wp, C = ref.shape
    z_row = jnp.zeros((B, 1, Wp, C), ref.dtype)
    ref[:, 0:1, :, :] = z_row
    ref[:, H + 1:H + 2, :, :] = z_row
    ref[:, :, W:Wp, :] = jnp.zeros((B, Hp, Wp - W, C), ref.dtype)


def _conv_block(src, w_ref, b_ref, H, strip, pool, emit):
    B = src.shape[0]
    Wp = src.shape[2]
    Cin = src.shape[-1]
    Cout = w_ref.shape[-1]
    bias = b_ref[...]
    M = B * strip * Wp
    for h0 in range(0, H, strip):
        acc = jnp.zeros((M, Cout), jnp.float32)
        for kh in range(3):
            base = src[:, h0 + kh:h0 + kh + strip, :, :].reshape(M, Cin)
            acc = acc + jnp.dot(jnp.roll(base, 1, axis=0), w_ref[kh * 3],
                                preferred_element_type=jnp.float32)
            acc = acc + jnp.dot(base, w_ref[kh * 3 + 1],
                                preferred_element_type=jnp.float32)
            acc = acc + jnp.dot(jnp.roll(base, -1, axis=0), w_ref[kh * 3 + 2],
                                preferred_element_type=jnp.float32)
        act = jnp.maximum(acc + bias, 0.0)
        a = act.astype(jnp.bfloat16)
        if pool:
            a = a.reshape(B, strip, Wp // 2, 2, Cout)
            a = jnp.maximum(a[:, :, :, 0, :], a[:, :, :, 1, :])
            a = a.reshape(B, strip // 2, 2, Wp // 2, Cout)
            a = jnp.maximum(a[:, :, 0], a[:, :, 1])
        else:
            a = a.reshape(B, strip, Wp, Cout)
        emit(h0, a)


def _store_interior(dst, h0, a):
    s, Wv = a.shape[1], a.shape[2]
    dst[:, 1 + h0:1 + h0 + s, 0:Wv, :] = a


def _store_split(dst, h0, a):
    B, s, Wv, C2 = a.shape
    C = C2 // 2
    dst[0:B, 1 + h0:1 + h0 + s, 0:Wv, :] = a[..., :C]
    dst[B:2 * B, 1 + h0:1 + h0 + s, 0:Wv, :] = a[..., C:]


def _conv_stack_kernel(xp_ref, w0, b0, w1, b1, w2, b2, w3, b3, w4, b4,
                       w5, b5, feats_ref, s0, s1, s2, s3, s4):
    _conv_block(xp_ref, w0, b0, 32, 2, False,
                lambda h0, a: _store_interior(s0, h0, a))
    _zero_halo(s0, 32, 32)
    _conv_block(s0, w1, b1, 32, 2, True,
                lambda h0, a: _store_interior(s1, h0 // 2, a))
    _zero_halo(s1, 16, 16)
    _conv_block(s1, w2, b2, 16, 2, False,
                lambda h0, a: _store_split(s2, h0, a))
    _zero_halo(s2, 16, 16)
    _conv_block(s2, w3, b3, 16, 2, True,
                lambda h0, a: _store_split(s3, h0 // 2, a))
    _zero_halo(s3, 8, 8)
    _conv_block(s3, w4, b4, 8, 2, False,
                lambda h0, a: _store_interior(s4, h0, a))
    _zero_halo(s4, 8, 8)

    def emit5(h0, a):
        B, s, _, C = a.shape
        ho = h0 // 2
        a = a[:, :, 0:4, :]
        feats_ref[:, ho * 4:(ho + s) * 4, :] = a.reshape(B, s * 4, C)

    _conv_block(s4, w5, b5, 8, 2, True, emit5)


def _fc_kernel(a_ref, w1_ref, b1_ref, w2_ref, b2_ref, o_ref):
    h = jnp.dot(a_ref[...], w1_ref[...], preferred_element_type=jnp.float32)
    h = jnp.maximum(h + b1_ref[...], 0.0).astype(jnp.bfloat16)
    o = jnp.dot(h, w2_ref[...], preferred_element_type=jnp.float32)
    o_ref[...] = o + b2_ref[...]


def _block_diag(w, k):
    eye = jnp.eye(k, dtype=w.dtype)
    t, ci, co = w.shape
    return jnp.einsum('jk,tco->tjcko', eye, w).reshape(t, k * ci, k * co)


def kernel(x, conv_w0, conv_b0, conv_w1, conv_b1, conv_w2, conv_b2,
           conv_w3, conv_b3, conv_w4, conv_b4, conv_w5, conv_b5,
           hidden_w0, hidden_b0, out_w, out_b):
    N = x.shape[0]
    Q = N // 4
    bf = jnp.bfloat16

    x_pack = (x.reshape(4, Q, 3, 32, 32).transpose(1, 3, 4, 0, 2)
              .reshape(Q, 32, 32, 12))
    xp = jnp.pad(x_pack, ((0, 0), (1, 1), (0, 16), (0, 4))).astype(bf)

    ws = [jnp.pad(_block_diag(conv_w0, 4), ((0, 0), (0, 4), (0, 0))),
          _block_diag(conv_w1, 4), _block_diag(conv_w2, 4),
          _block_diag(conv_w3, 2), conv_w4, conv_w5]
    ws = [w.astype(bf) for w in ws]
    bs = [jnp.tile(conv_b0, 4), jnp.tile(conv_b1, 4), jnp.tile(conv_b2, 4),
          jnp.tile(conv_b3, 2), conv_b4, conv_b5]
    bs = [b.reshape(1, -1) for b in bs]

    wspecs = []
    operands = [xp]
    for w, b in zip(ws, bs):
        operands += [w, b]
        wspecs += [pl.BlockSpec(w.shape, lambda n: (0, 0, 0)),
                   pl.BlockSpec(b.shape, lambda n: (0, 0))]

    feats = pl.pallas_call(
        _conv_stack_kernel,
        out_shape=jax.ShapeDtypeStruct((N, 16, 256), bf),
        grid_spec=pltpu.PrefetchScalarGridSpec(
            num_scalar_prefetch=0,
            grid=(Q // 4,),
            in_specs=[pl.BlockSpec((4, 34, 48, 16),
                                   lambda n: (n, 0, 0, 0))] + wspecs,
            out_specs=pl.BlockSpec((_BN, 16, 256), lambda n: (n, 0, 0)),
            scratch_shapes=[
                pltpu.VMEM((4, 34, 48, 256), bf),
                pltpu.VMEM((4, 18, 32, 256), bf),
                pltpu.VMEM((8, 18, 32, 256), bf),
                pltpu.VMEM((16, 10, 16, 128), bf),
                pltpu.VMEM((16, 10, 16, 256), bf),
            ],
        ),
        compiler_params=pltpu.CompilerParams(
            dimension_semantics=("parallel",),
            vmem_limit_bytes=_VMEM_LIMIT,
        ),
    )(*operands)

    feats = feats.reshape(N // 16, 4, 4, 16 * 256)
    feats = feats[:, jnp.array([0, 2, 1, 3]), :, :]
    feats = jnp.transpose(feats, (1, 0, 2, 3)).reshape(N, 4096)

    w1p = (hidden_w0.reshape(256, 4, 4, 512).transpose(1, 2, 0, 3)
           .reshape(4096, 512).astype(bf))

    out = pl.pallas_call(
        _fc_kernel,
        out_shape=jax.ShapeDtypeStruct((N, 100), jnp.float32),
        grid_spec=pltpu.PrefetchScalarGridSpec(
            num_scalar_prefetch=0,
            grid=(4,),
            in_specs=[
                pl.BlockSpec((N // 4, 4096), lambda n: (n, 0)),
                pl.BlockSpec((4096, 512), lambda n: (0, 0)),
                pl.BlockSpec((1, 512), lambda n: (0, 0)),
                pl.BlockSpec((512, 100), lambda n: (0, 0)),
                pl.BlockSpec((1, 100), lambda n: (0, 0)),
            ],
            out_specs=pl.BlockSpec((N // 4, 100), lambda n: (n, 0)),
        ),
        compiler_params=pltpu.CompilerParams(
            dimension_semantics=("parallel",),
            vmem_limit_bytes=_VMEM_LIMIT,
        ),
    )(feats, w1p, hidden_b0.reshape(1, 512), out_w.astype(bf),
      out_b.reshape(1, 100))
    return out


# speedup vs baseline: 1.4629x; 1.4629x over previous
import functools

import jax
import jax.numpy as jnp
from jax.experimental import pallas as pl
from jax.experimental.pallas import tpu as pltpu

_VMEM_LIMIT = 64 * 1024 * 1024
_BN = 16


def _zero_border(ref, H, W):
    B, Hp, Wp, C = ref.shape
    z_row = jnp.zeros((B, 1, Wp, C), ref.dtype)
    z_col = jnp.zeros((B, Hp, Wp - W - 1, C), ref.dtype)
    ref[:, 0:1, :, :] = z_row
    ref[:, H + 1:H + 2, :, :] = z_row
    ref[:, :, 0:1, :] = z_col[:, :, 0:1, :]
    ref[:, :, W + 1:Wp, :] = z_col


def _conv_block(src, w_ref, b_ref, H, W, strip, pool, emit):
    B = src.shape[0]
    Cin = src.shape[-1]
    Cout = w_ref.shape[-1]
    bias = b_ref[...]
    for h0 in range(0, H, strip):
        acc = jnp.zeros((B * strip * W, Cout), jnp.float32)
        for kh in range(3):
            for kw in range(3):
                tap = src[:, h0 + kh:h0 + kh + strip, kw:kw + W, :]
                tap = tap.reshape(B * strip * W, Cin)
                acc = acc + jnp.dot(tap, w_ref[kh * 3 + kw],
                                    preferred_element_type=jnp.float32)
        act = jnp.maximum(acc + bias, 0.0)
        a = act.astype(jnp.bfloat16)
        if pool:
            a = a.reshape(B, strip, W // 2, 2, Cout)
            a = jnp.maximum(a[:, :, :, 0, :], a[:, :, :, 1, :])
            a = a.reshape(B, strip // 2, 2, W // 2, Cout)
            a = jnp.maximum(a[:, :, 0], a[:, :, 1])
        else:
            a = a.reshape(B, strip, W, Cout)
        emit(h0, a)


def _store_interior(dst, h0, a):
    s = a.shape[1]
    W = a.shape[2]
    dst[:, 1 + h0:1 + h0 + s, 1:1 + W, :] = a


def _store_split(dst, h0, a):
    B, s, W, C2 = a.shape
    C = C2 // 2
    dst[0:B, 1 + h0:1 + h0 + s, 1:1 + W, :] = a[..., :C]
    dst[B:2 * B, 1 + h0:1 + h0 + s, 1:1 + W, :] = a[..., C:]


def _conv_stack_kernel(xp_ref, w0, b0, w1, b1, w2, b2, w3, b3, w4, b4,
                       w5, b5, feats_ref, s0, s1, s2, s3, s4):
    for s, (H, W) in ((s0, (32, 32)), (s1, (16, 16)), (s2, (16, 16)),
                      (s3, (8, 8)), (s4, (8, 8))):
        _zero_border(s, H, W)

    _conv_block(xp_ref, w0, b0, 32, 32, 2, False,
                lambda h0, a: _store_interior(s0, h0, a))
    _conv_block(s0, w1, b1, 32, 32, 2, True,
                lambda h0, a: _store_interior(s1, h0 // 2, a))
    _conv_block(s1, w2, b2, 16, 16, 2, False,
                lambda h0, a: _store_split(s2, h0, a))
    _conv_block(s2, w3, b3, 16, 16, 2, True,
                lambda h0, a: _store_split(s3, h0 // 2, a))
    _conv_block(s3, w4, b4, 8, 8, 2, False,
                lambda h0, a: _store_interior(s4, h0, a))

    def emit5(h0, a):
        B, s, W2, C = a.shape
        ho = h0 // 2
        feats_ref[:, ho * 4:(ho + s) * 4, :] = a.reshape(B, s * W2, C)

    _conv_block(s4, w5, b5, 8, 8, 2, True, emit5)


def _fc_kernel(a_ref, w1_ref, b1_ref, w2_ref, b2_ref, o_ref):
    h = jnp.dot(a_ref[...], w1_ref[...], preferred_element_type=jnp.float32)
    h = jnp.maximum(h + b1_ref[...], 0.0).astype(jnp.bfloat16)
    o = jnp.dot(h, w2_ref[...], preferred_element_type=jnp.float32)
    o_ref[...] = o + b2_ref[...]


def _block_diag(w, k):
    eye = jnp.eye(k, dtype=w.dtype)
    t, ci, co = w.shape
    return jnp.einsum('jk,tco->tjcko', eye, w).reshape(t, k * ci, k * co)


def kernel(x, conv_w0, conv_b0, conv_w1, conv_b1, conv_w2, conv_b2,
           conv_w3, conv_b3, conv_w4, conv_b4, conv_w5, conv_b5,
           hidden_w0, hidden_b0, out_w, out_b):
    N = x.shape[0]
    Q = N // 4
    bf = jnp.bfloat16

    x_nhwc = jnp.transpose(x, (0, 2, 3, 1)).reshape(4, Q, 32, 32, 3)
    x_pack = jnp.transpose(x_nhwc, (1, 2, 3, 0, 4)).reshape(Q, 32, 32, 12)
    xp = jnp.pad(x_pack, ((0, 0), (1, 1), (1, 15), (0, 0))).astype(bf)

    ws = [_block_diag(conv_w0, 4), _block_diag(conv_w1, 4),
          _block_diag(conv_w2, 4), _block_diag(conv_w3, 2),
          conv_w4, conv_w5]
    ws = [w.astype(bf) for w in ws]
    bs = [jnp.tile(conv_b0, 4), jnp.tile(conv_b1, 4), jnp.tile(conv_b2, 4),
          jnp.tile(conv_b3, 2), conv_b4, conv_b5]
    bs = [b.reshape(1, -1) for b in bs]

    wspecs = []
    operands = [xp]
    for w, b in zip(ws, bs):
        operands += [w, b]
        wspecs += [pl.BlockSpec(w.shape, lambda n: (0, 0, 0)),
                   pl.BlockSpec(b.shape, lambda n: (0, 0))]

    feats = pl.pallas_call(
        _conv_stack_kernel,
        out_shape=jax.ShapeDtypeStruct((N, 16, 256), bf),
        grid_spec=pltpu.PrefetchScalarGridSpec(
            num_scalar_prefetch=0,
            grid=(Q // 4,),
            in_specs=[pl.BlockSpec((4, 34, 48, 12),
                                   lambda n: (n, 0, 0, 0))] + wspecs,
            out_specs=pl.BlockSpec((_BN, 16, 256), lambda n: (n, 0, 0)),
            scratch_shapes=[
                pltpu.VMEM((4, 34, 48, 256), bf),
                pltpu.VMEM((4, 18, 32, 256), bf),
                pltpu.VMEM((8, 18, 32, 256), bf),
                pltpu.VMEM((16, 10, 16, 128), bf),
                pltpu.VMEM((16, 10, 16, 256), bf),
            ],
        ),
        compiler_params=pltpu.CompilerParams(
            dimension_semantics=("parallel",),
            vmem_limit_bytes=_VMEM_LIMIT,
        ),
    )(*operands)

    feats = feats.reshape(N // 16, 4, 4, 16 * 256)
    feats = feats[:, jnp.array([0, 2, 1, 3]), :, :]
    feats = jnp.transpose(feats, (1, 0, 2, 3)).reshape(N, 4096)

    w1p = (hidden_w0.reshape(256, 4, 4, 512).transpose(1, 2, 0, 3)
           .reshape(4096, 512).astype(bf))

    out = pl.pallas_call(
        _fc_kernel,
        out_shape=jax.ShapeDtypeStruct((N, 100), jnp.float32),
        grid_spec=pltpu.PrefetchScalarGridSpec(
            num_scalar_prefetch=0,
            grid=(4,),
            in_specs=[
                pl.BlockSpec((N // 4, 4096), lambda n: (n, 0)),
                pl.BlockSpec((4096, 512), lambda n: (0, 0)),
                pl.BlockSpec((1, 512), lambda n: (0, 0)),
                pl.BlockSpec((512, 100), lambda n: (0, 0)),
                pl.BlockSpec((1, 100), lambda n: (0, 0)),
            ],
            out_specs=pl.BlockSpec((N // 4, 100), lambda n: (n, 0)),
        ),
        compiler_params=pltpu.CompilerParams(
            dimension_semantics=("parallel",),
            vmem_limit_bytes=_VMEM_LIMIT,
        ),
    )(feats, w1p, hidden_b0.reshape(1, 512), out_w.astype(bf),
      out_b.reshape(1, 100))
    return out
